# Initial kernel scaffold; baseline (speedup 1.0000x reference)
import jax
import jax.numpy as jnp
from jax.experimental import pallas as pl


def kernel(spikes, membrane):
    raise NotImplementedError("write your pallas kernel here")


# speedup vs baseline: 1.7720x; 1.7720x over previous
import jax
import jax.numpy as jnp
from jax import lax
from jax.experimental import pallas as pl
from jax.experimental.pallas import tpu as pltpu
from jax.experimental.pallas import tpu_sc as plsc

B, C, H, W = 4, 96, 224, 224
HW = H * W
NMAPS = B * C
NC, NS, LANES = 2, 16, 16
NW = NC * NS
MAPS_PER_W = NMAPS // NW
NCHUNK = 4
RW = H // NCHUNK
NSEG = W // LANES

NEG_INF = float("-inf")


def _wta_body(spk_hbm, mem_hbm, ospk_hbm, omem_hbm,
              sbuf0, sbuf1, mbuf0, mbuf1, zbuf, ohbuf,
              sem_in, sem_out, sem_p):
    wid = lax.axis_index("s") * NC + lax.axis_index("c")
    iota16 = lax.iota(jnp.int32, LANES)

    @pl.loop(0, RW)
    def _zinit(r):
        for seg in range(NSEG):
            zbuf[r, pl.ds(seg * LANES, LANES)] = jnp.zeros((LANES,),
                                                           jnp.float32)

    @pl.loop(0, MAPS_PER_W)
    def _per_map(m):
        map_id = wid * MAPS_PER_W + m
        b = map_id // C
        ch = map_id % C

        zcopies = []
        for k in range(NCHUNK):
            zcopies.append(pltpu.async_copy(
                zbuf, ospk_hbm.at[b, ch, pl.ds(k * RW, RW)], sem_out))
            zcopies.append(pltpu.async_copy(
                zbuf, omem_hbm.at[b, ch, pl.ds(k * RW, RW)], sem_out))

        bufs = ((sbuf0, mbuf0), (sbuf1, mbuf1))
        cur = (pltpu.async_copy(spk_hbm.at[b, ch, pl.ds(0, RW)],
                                sbuf0, sem_in),
               pltpu.async_copy(mem_hbm.at[b, ch, pl.ds(0, RW)],
                                mbuf0, sem_in))

        vmax = jnp.full((LANES,), NEG_INF, jnp.float32)
        vidx = jnp.zeros((LANES,), jnp.int32)
        for c4 in range(NCHUNK):
            sb, mb = bufs[c4 % 2]
            if c4 + 1 < NCHUNK:
                nsb, nmb = bufs[(c4 + 1) % 2]
                nxt = (pltpu.async_copy(
                           spk_hbm.at[b, ch, pl.ds((c4 + 1) * RW, RW)],
                           nsb, sem_in),
                       pltpu.async_copy(
                           mem_hbm.at[b, ch, pl.ds((c4 + 1) * RW, RW)],
                           nmb, sem_in))
            cur[0].wait()
            cur[1].wait()
            row0_flat = c4 * RW * W

            def step(r, carry, sb=sb, mb=mb, row0_flat=row0_flat):
                vm, vi = carry
                rflat = row0_flat + r * W
                for seg in range(NSEG):
                    s = sb[r, pl.ds(seg * LANES, LANES)]
                    mv = mb[r, pl.ds(seg * LANES, LANES)]
                    masked = jnp.where(s > 0.0, mv, NEG_INF)
                    upd = masked > vm
                    idxv = iota16 + (rflat + seg * LANES)
                    vm = jnp.where(upd, masked, vm)
                    vi = jnp.where(upd, idxv, vi)
                return (vm, vi)

            vmax, vidx = lax.fori_loop(0, RW, step, (vmax, vidx))
            if c4 + 1 < NCHUNK:
                cur = nxt

        mval = jnp.float32(NEG_INF)
        widx = jnp.int32(HW)
        for j in range(LANES):
            v = vmax[j]
            i = vidx[j]
            upd = (v > mval) | ((v == mval) & (i < widx))
            mval = jnp.where(upd, v, mval)
            widx = jnp.where(upd, i, widx)
        has = mval != NEG_INF

        r_w = widx // W
        c_w = widx - r_w * W
        c0 = (c_w // LANES) * LANES
        lane = c_w - c0
        oneval = jnp.where(has, 1.0, 0.0).astype(jnp.float32)
        ohbuf[:] = jnp.where(iota16 == lane, oneval,
                             jnp.zeros((), jnp.float32))

        for d in zcopies:
            d.wait()

        @pl.when(has)
        def _patch():
            pltpu.async_copy(
                ohbuf, ospk_hbm.at[b, ch, r_w, pl.ds(c0, LANES)],
                sem_p).wait()

        @pl.when(jnp.logical_not(has))
        def _restore_membrane():
            for k in range(NCHUNK):
                pltpu.async_copy(
                    mem_hbm.at[b, ch, pl.ds(k * RW, RW)], mbuf0,
                    sem_in).wait()
                pltpu.async_copy(
                    mbuf0, omem_hbm.at[b, ch, pl.ds(k * RW, RW)],
                    sem_p).wait()


_wta = pl.kernel(
    _wta_body,
    out_type=(jax.ShapeDtypeStruct((B, C, H, W), jnp.float32),
              jax.ShapeDtypeStruct((B, C, H, W), jnp.float32)),
    mesh=plsc.VectorSubcoreMesh(
        core_axis_name="c", subcore_axis_name="s",
        num_cores=NC, num_subcores=NS),
    scratch_types=[
        pltpu.VMEM((RW, W), jnp.float32),
        pltpu.VMEM((RW, W), jnp.float32),
        pltpu.VMEM((RW, W), jnp.float32),
        pltpu.VMEM((RW, W), jnp.float32),
        pltpu.VMEM((RW, W), jnp.float32),
        pltpu.VMEM((LANES,), jnp.float32),
        pltpu.SemaphoreType.DMA,
        pltpu.SemaphoreType.DMA,
        pltpu.SemaphoreType.DMA,
    ],
)


@jax.jit
def kernel(spikes, membrane):
    new_spikes, new_membrane = _wta(spikes, membrane)
    return (new_spikes, new_membrane, new_spikes)


# speedup vs baseline: 1.9994x; 1.1283x over previous
import jax
import jax.numpy as jnp
from jax import lax
from jax.experimental import pallas as pl
from jax.experimental.pallas import tpu as pltpu
from jax.experimental.pallas import tpu_sc as plsc

B, C, H, W = 4, 96, 224, 224
HW = H * W
NMAPS = B * C
NC, NS, LANES = 2, 16, 16
NW = NC * NS
MAPS_PER_W = NMAPS // NW
NCHUNK = 4
RW = H // NCHUNK
NSEG = W // LANES

NEG_INF = float("-inf")


def _wta_body(spk_hbm, mem_hbm, ospk_hbm, omem_hbm, omask_hbm,
              sbuf0, sbuf1, mbuf0, mbuf1, zbuf, obuf, smem,
              sem_in, sem_out, sem_p):
    wid = lax.axis_index("s") * NC + lax.axis_index("c")
    iota16 = lax.iota(jnp.int32, LANES)
    zeros16 = jnp.zeros((LANES,), jnp.float32)

    @pl.loop(0, RW)
    def _zinit(r):
        for seg in range(NSEG):
            zbuf[r, pl.ds(seg * LANES, LANES)] = zeros16
            obuf[r, pl.ds(seg * LANES, LANES)] = zeros16

    smem[2] = jnp.int32(0)
    smem[3] = jnp.int32(0)

    def scan_map(map_id):
        b = map_id // C
        ch = map_id % C
        bufs = ((sbuf0, mbuf0), (sbuf1, mbuf1))
        cur = (pltpu.async_copy(spk_hbm.at[b, ch, pl.ds(0, RW)],
                                sbuf0, sem_in),
               pltpu.async_copy(mem_hbm.at[b, ch, pl.ds(0, RW)],
                                mbuf0, sem_in))
        vmax = jnp.full((LANES,), NEG_INF, jnp.float32)
        vidx = jnp.zeros((LANES,), jnp.int32)
        for c4 in range(NCHUNK):
            sb, mb = bufs[c4 % 2]
            if c4 + 1 < NCHUNK:
                nsb, nmb = bufs[(c4 + 1) % 2]
                nxt = (pltpu.async_copy(
                           spk_hbm.at[b, ch, pl.ds((c4 + 1) * RW, RW)],
                           nsb, sem_in),
                       pltpu.async_copy(
                           mem_hbm.at[b, ch, pl.ds((c4 + 1) * RW, RW)],
                           nmb, sem_in))
            cur[0].wait()
            cur[1].wait()
            row0_flat = c4 * RW * W

            def step(r, carry, sb=sb, mb=mb, row0_flat=row0_flat):
                vm, vi = carry
                rflat = row0_flat + r * W
                for seg in range(NSEG):
                    s = sb[r, pl.ds(seg * LANES, LANES)]
                    mv = mb[r, pl.ds(seg * LANES, LANES)]
                    masked = jnp.where(s > 0.0, mv, NEG_INF)
                    upd = masked > vm
                    idxv = iota16 + (rflat + seg * LANES)
                    vm = jnp.where(upd, masked, vm)
                    vi = jnp.where(upd, idxv, vi)
                return (vm, vi)

            vmax, vidx = lax.fori_loop(0, RW, step, (vmax, vidx))
            if c4 + 1 < NCHUNK:
                cur = nxt

        mval = jnp.float32(NEG_INF)
        widx = jnp.int32(HW)
        for j in range(LANES):
            v = vmax[j]
            i = vidx[j]
            upd = (v > mval) | ((v == mval) & (i < widx))
            mval = jnp.where(upd, v, mval)
            widx = jnp.where(upd, i, widx)
        has = mval != NEG_INF
        smem[0] = jnp.where(has, jnp.int32(1), jnp.int32(0))
        smem[1] = jnp.where(has, widx, jnp.int32(0))

    def write_map(map_id):
        b = map_id // C
        ch = map_id % C
        has = smem[0] == 1
        widx = smem[1]
        sr = smem[2]
        sc = pl.multiple_of(smem[3], LANES)
        obuf[sr, pl.ds(sc, LANES)] = zeros16
        cw = widx // (RW * W)
        rr = widx // W
        r_loc = rr - cw * RW
        c_w = widx - rr * W
        c0 = pl.multiple_of((c_w // LANES) * LANES, LANES)
        lane = c_w - c0
        oneval = jnp.where(has, 1.0, 0.0).astype(jnp.float32)
        obuf[r_loc, pl.ds(c0, LANES)] = jnp.where(iota16 == lane, oneval,
                                                  jnp.zeros((), jnp.float32))
        smem[2] = r_loc
        smem[3] = c0

        wcopies = []
        for j in range(NCHUNK - 1):
            kj = j + jnp.where(j >= cw, 1, 0)
            wcopies.append(pltpu.async_copy(
                zbuf, ospk_hbm.at[b, ch, pl.ds(kj * RW, RW)], sem_out))
            wcopies.append(pltpu.async_copy(
                zbuf, omask_hbm.at[b, ch, pl.ds(kj * RW, RW)], sem_out))
        wcopies.append(pltpu.async_copy(
            obuf, ospk_hbm.at[b, ch, pl.ds(cw * RW, RW)], sem_out))
        wcopies.append(pltpu.async_copy(
            obuf, omask_hbm.at[b, ch, pl.ds(cw * RW, RW)], sem_out))

        mcopies = []

        @pl.when(has)
        def _zero_membrane():
            for k in range(NCHUNK):
                mcopies.append(pltpu.async_copy(
                    zbuf, omem_hbm.at[b, ch, pl.ds(k * RW, RW)], sem_out))

        @pl.when(jnp.logical_not(has))
        def _restore_membrane():
            for k in range(NCHUNK):
                pltpu.async_copy(
                    mem_hbm.at[b, ch, pl.ds(k * RW, RW)], mbuf0,
                    sem_in).wait()
                pltpu.async_copy(
                    mbuf0, omem_hbm.at[b, ch, pl.ds(k * RW, RW)],
                    sem_p).wait()

        return wcopies, has, mcopies

    scan_map(wid * MAPS_PER_W)

    @pl.loop(1, MAPS_PER_W + 1)
    def _pipe(m):
        wcopies, has, mcopies = write_map(wid * MAPS_PER_W + m - 1)

        @pl.when(m < MAPS_PER_W)
        def _scan_next():
            scan_map(wid * MAPS_PER_W + m)

        for d in wcopies:
            d.wait()

        @pl.when(has)
        def _drain_membrane_zeros():
            for d in mcopies:
                d.wait()


_wta = pl.kernel(
    _wta_body,
    out_type=(jax.ShapeDtypeStruct((B, C, H, W), jnp.float32),
              jax.ShapeDtypeStruct((B, C, H, W), jnp.float32),
              jax.ShapeDtypeStruct((B, C, H, W), jnp.float32)),
    mesh=plsc.VectorSubcoreMesh(
        core_axis_name="c", subcore_axis_name="s",
        num_cores=NC, num_subcores=NS),
    scratch_types=[
        pltpu.VMEM((RW, W), jnp.float32),
        pltpu.VMEM((RW, W), jnp.float32),
        pltpu.VMEM((RW, W), jnp.float32),
        pltpu.VMEM((RW, W), jnp.float32),
        pltpu.VMEM((RW, W), jnp.float32),
        pltpu.VMEM((RW, W), jnp.float32),
        pltpu.SMEM((8,), jnp.int32),
        pltpu.SemaphoreType.DMA,
        pltpu.SemaphoreType.DMA,
        pltpu.SemaphoreType.DMA,
    ],
)


@jax.jit
def kernel(spikes, membrane):
    new_spikes, new_membrane, winner_mask = _wta(spikes, membrane)
    return (new_spikes, new_membrane, winner_mask)


# speedup vs baseline: 2.0001x; 1.0004x over previous
import jax
import jax.numpy as jnp
from jax import lax
from jax.experimental import pallas as pl
from jax.experimental.pallas import tpu as pltpu
from jax.experimental.pallas import tpu_sc as plsc

B, C, H, W = 4, 96, 224, 224
HW = H * W
NMAPS = B * C
NC, NS, LANES = 2, 16, 16
NW = NC * NS
MAPS_PER_W = NMAPS // NW
NCHUNK = 4
RW = H // NCHUNK
NSEG = W // LANES

NEG_INF = float("-inf")


def _wta_body(spk_hbm, mem_hbm, ospk_hbm, omem_hbm, omask_hbm,
              sbuf0, sbuf1, mbuf0, mbuf1, zbuf, obuf, smem,
              sem_in, sem_out, sem_p):
    wid = lax.axis_index("s") * NC + lax.axis_index("c")
    iota16 = lax.iota(jnp.int32, LANES)
    zeros16 = jnp.zeros((LANES,), jnp.float32)

    @pl.loop(0, RW)
    def _zinit(r):
        for seg in range(NSEG):
            zbuf[r, pl.ds(seg * LANES, LANES)] = zeros16
            obuf[r, pl.ds(seg * LANES, LANES)] = zeros16

    smem[2] = jnp.int32(0)
    smem[3] = jnp.int32(0)

    def scan_map(map_id, next_map_id):
        b = map_id // C
        ch = map_id % C
        bufs = ((sbuf0, mbuf0), (sbuf1, mbuf1))
        cur = (pltpu.make_async_copy(spk_hbm.at[b, ch, pl.ds(0, RW)],
                                     sbuf0, sem_in),
               pltpu.make_async_copy(mem_hbm.at[b, ch, pl.ds(0, RW)],
                                     mbuf0, sem_in))
        vmax = jnp.full((LANES,), NEG_INF, jnp.float32)
        vidx = jnp.zeros((LANES,), jnp.int32)
        for c4 in range(NCHUNK):
            sb, mb = bufs[c4 % 2]
            if c4 + 1 < NCHUNK:
                nsb, nmb = bufs[(c4 + 1) % 2]
                nxt = (pltpu.async_copy(
                           spk_hbm.at[b, ch, pl.ds((c4 + 1) * RW, RW)],
                           nsb, sem_in),
                       pltpu.async_copy(
                           mem_hbm.at[b, ch, pl.ds((c4 + 1) * RW, RW)],
                           nmb, sem_in))
            else:
                nb = next_map_id // C
                nch = next_map_id % C
                pltpu.async_copy(spk_hbm.at[nb, nch, pl.ds(0, RW)],
                                 sbuf0, sem_in)
                pltpu.async_copy(mem_hbm.at[nb, nch, pl.ds(0, RW)],
                                 mbuf0, sem_in)
            cur[0].wait()
            cur[1].wait()
            row0_flat = c4 * RW * W

            def step(r, carry, sb=sb, mb=mb, row0_flat=row0_flat):
                vm, vi = carry
                rflat = row0_flat + r * W
                for seg in range(NSEG):
                    s = sb[r, pl.ds(seg * LANES, LANES)]
                    mv = mb[r, pl.ds(seg * LANES, LANES)]
                    masked = jnp.where(s > 0.0, mv, NEG_INF)
                    upd = masked > vm
                    idxv = iota16 + (rflat + seg * LANES)
                    vm = jnp.where(upd, masked, vm)
                    vi = jnp.where(upd, idxv, vi)
                return (vm, vi)

            vmax, vidx = lax.fori_loop(0, RW, step, (vmax, vidx))
            if c4 + 1 < NCHUNK:
                cur = nxt

        mval = jnp.float32(NEG_INF)
        widx = jnp.int32(HW)
        for j in range(LANES):
            v = vmax[j]
            i = vidx[j]
            upd = (v > mval) | ((v == mval) & (i < widx))
            mval = jnp.where(upd, v, mval)
            widx = jnp.where(upd, i, widx)
        has = mval != NEG_INF
        smem[0] = jnp.where(has, jnp.int32(1), jnp.int32(0))
        smem[1] = jnp.where(has, widx, jnp.int32(0))

    def write_map(map_id):
        b = map_id // C
        ch = map_id % C
        has = smem[0] == 1
        widx = smem[1]
        sr = smem[2]
        sc = pl.multiple_of(smem[3], LANES)
        obuf[sr, pl.ds(sc, LANES)] = zeros16
        cw = widx // (RW * W)
        rr = widx // W
        r_loc = rr - cw * RW
        c_w = widx - rr * W
        c0 = pl.multiple_of((c_w // LANES) * LANES, LANES)
        lane = c_w - c0
        oneval = jnp.where(has, 1.0, 0.0).astype(jnp.float32)
        obuf[r_loc, pl.ds(c0, LANES)] = jnp.where(iota16 == lane, oneval,
                                                  jnp.zeros((), jnp.float32))
        smem[2] = r_loc
        smem[3] = c0

        wcopies = []
        for j in range(NCHUNK - 1):
            kj = j + jnp.where(j >= cw, 1, 0)
            wcopies.append(pltpu.async_copy(
                zbuf, ospk_hbm.at[b, ch, pl.ds(kj * RW, RW)], sem_out))
            wcopies.append(pltpu.async_copy(
                zbuf, omask_hbm.at[b, ch, pl.ds(kj * RW, RW)], sem_out))
        wcopies.append(pltpu.async_copy(
            obuf, ospk_hbm.at[b, ch, pl.ds(cw * RW, RW)], sem_out))
        wcopies.append(pltpu.async_copy(
            obuf, omask_hbm.at[b, ch, pl.ds(cw * RW, RW)], sem_out))

        mcopies = []

        @pl.when(has)
        def _zero_membrane():
            for k in range(NCHUNK):
                mcopies.append(pltpu.async_copy(
                    zbuf, omem_hbm.at[b, ch, pl.ds(k * RW, RW)], sem_out))

        @pl.when(jnp.logical_not(has))
        def _restore_membrane():
            for k in range(NCHUNK):
                pltpu.async_copy(
                    mem_hbm.at[b, ch, pl.ds(k * RW, RW)], mbuf1,
                    sem_p).wait()
                pltpu.async_copy(
                    mbuf1, omem_hbm.at[b, ch, pl.ds(k * RW, RW)],
                    sem_p).wait()

        return wcopies, has, mcopies

    base = wid * MAPS_PER_W
    last = base + MAPS_PER_W - 1
    pltpu.async_copy(spk_hbm.at[base // C, base % C, pl.ds(0, RW)],
                     sbuf0, sem_in)
    pltpu.async_copy(mem_hbm.at[base // C, base % C, pl.ds(0, RW)],
                     mbuf0, sem_in)
    scan_map(base, base + 1)

    @pl.loop(1, MAPS_PER_W + 1)
    def _pipe(m):
        wcopies, has, mcopies = write_map(base + m - 1)

        @pl.when(m < MAPS_PER_W)
        def _scan_next():
            scan_map(base + m, base + jnp.minimum(m + 1, MAPS_PER_W - 1))

        for d in wcopies:
            d.wait()

        @pl.when(has)
        def _drain_membrane_zeros():
            for d in mcopies:
                d.wait()

    pltpu.make_async_copy(spk_hbm.at[last // C, last % C, pl.ds(0, RW)],
                          sbuf0, sem_in).wait()
    pltpu.make_async_copy(mem_hbm.at[last // C, last % C, pl.ds(0, RW)],
                          mbuf0, sem_in).wait()


_wta = pl.kernel(
    _wta_body,
    out_type=(jax.ShapeDtypeStruct((B, C, H, W), jnp.float32),
              jax.ShapeDtypeStruct((B, C, H, W), jnp.float32),
              jax.ShapeDtypeStruct((B, C, H, W), jnp.float32)),
    mesh=plsc.VectorSubcoreMesh(
        core_axis_name="c", subcore_axis_name="s",
        num_cores=NC, num_subcores=NS),
    scratch_types=[
        pltpu.VMEM((RW, W), jnp.float32),
        pltpu.VMEM((RW, W), jnp.float32),
        pltpu.VMEM((RW, W), jnp.float32),
        pltpu.VMEM((RW, W), jnp.float32),
        pltpu.VMEM((RW, W), jnp.float32),
        pltpu.VMEM((RW, W), jnp.float32),
        pltpu.SMEM((8,), jnp.int32),
        pltpu.SemaphoreType.DMA,
        pltpu.SemaphoreType.DMA,
        pltpu.SemaphoreType.DMA,
    ],
)


@jax.jit
def kernel(spikes, membrane):
    new_spikes, new_membrane, winner_mask = _wta(spikes, membrane)
    return (new_spikes, new_membrane, winner_mask)
